# baseline (speedup 1.0000x reference)
import jax
import jax.numpy as jnp
from jax.experimental import pallas as pl


def _bn(x, g, b, eps):
    axes = tuple(i for i in range(x.ndim) if i != 1)
    m = jnp.mean(x, axis=axes, keepdims=True)
    v = jnp.var(x, axis=axes, keepdims=True)
    sh = [1] * x.ndim
    sh[1] = -1
    return g.reshape(sh) * (x - m) / jnp.sqrt(v + eps) + b.reshape(sh)


def _conv3d(x, w, b):
    y = jax.lax.conv_general_dilated(x, w, (1, 1, 1), 'SAME',
                                     dimension_numbers=('NCDHW', 'OIDHW', 'NCDHW'))
    return y + b.reshape(1, -1, 1, 1, 1)


def _voxelize(features, coords, r):
    B, C, N = features.shape
    norm = coords - jnp.mean(coords, axis=2, keepdims=True)
    scale = jnp.max(jnp.sqrt(jnp.sum(norm ** 2, axis=1, keepdims=True)), axis=2, keepdims=True) * 2.0
    norm = norm / scale + 0.5
    norm = jnp.clip(norm * r, 0.0, r - 1.0)
    vi = jnp.round(norm).astype(jnp.int32)
    flat = (vi[:, 0] * r + vi[:, 1]) * r + vi[:, 2]
    def scat(f, fl):
        sums = jax.ops.segment_sum(f.T, fl, num_segments=r * r * r)
        cnt = jax.ops.segment_sum(jnp.ones((fl.shape[0],), jnp.float32), fl, num_segments=r * r * r)
        return (sums / jnp.maximum(cnt, 1.0)[:, None]).T
    vox = jax.vmap(scat)(features, flat).reshape(B, C, r, r, r)
    return vox, norm


def _devoxelize(vox, norm, r):
    B, C = vox.shape[0], vox.shape[1]
    N = norm.shape[2]
    f = vox.reshape(B, C, r * r * r)
    def corners(v):
        v0 = jnp.floor(v)
        i0 = jnp.clip(v0.astype(jnp.int32), 0, r - 1)
        i1 = jnp.clip(i0 + 1, 0, r - 1)
        w1 = v - v0
        return ((i0, 1.0 - w1), (i1, w1))
    xs = corners(norm[:, 0])
    ys = corners(norm[:, 1])
    zs = corners(norm[:, 2])
    out = jnp.zeros((B, C, N), jnp.float32)
    for xi, wx in xs:
        for yi, wy in ys:
            for zi, wz in zs:
                flat = (xi * r + yi) * r + zi
                g = jnp.take_along_axis(f, jnp.broadcast_to(flat[:, None, :], (B, C, N)), axis=2)
                out = out + g * (wx * wy * wz)[:, None, :]
    return out


def _pvconv(p, features, coords, r):
    vox, norm = _voxelize(features, coords, r)
    h = _conv3d(vox, p['vw1'], p['vb1'])
    h = jax.nn.leaky_relu(_bn(h, p['g1'], p['be1'], 1e-4), 0.1)
    h = _conv3d(h, p['vw2'], p['vb2'])
    h = jax.nn.leaky_relu(_bn(h, p['g2'], p['be2'], 1e-4), 0.1)
    s = jnp.mean(h, axis=(2, 3, 4))
    s = jax.nn.relu(s @ p['sew1'].T + p['seb1'])
    s = jax.nn.sigmoid(s @ p['sew2'].T + p['seb2'])
    h = h * s[:, :, None, None, None]
    vfeat = _devoxelize(h, norm, r)
    pf = jnp.einsum('oc,bcn->bon', p['pw'], features) + p['pb'][None, :, None]
    pf = jax.nn.relu(_bn(pf, p['pg'], p['pbe'], 1e-5))
    return vfeat + pf


def _fps(coords, M):
    B, _, N = coords.shape
    pts = jnp.transpose(coords, (0, 2, 1))
    def body(i, state):
        idxs, dists, last = state
        lp = jnp.take_along_axis(pts, last[:, None, None], axis=1)
        d = jnp.sum((pts - lp) ** 2, axis=-1)
        dists = jnp.minimum(dists, d)
        nxt = jnp.argmax(dists, axis=-1).astype(jnp.int32)
        idxs = idxs.at[:, i].set(nxt)
        return (idxs, dists, nxt)
    state = (jnp.zeros((B, M), jnp.int32), jnp.full((B, N), 1e10, jnp.float32), jnp.zeros((B,), jnp.int32))
    return jax.lax.fori_loop(1, M, body, state)[0]


def _ball_query(coords, centers, radius, K):
    pts = jnp.transpose(coords, (0, 2, 1))
    ctr = jnp.transpose(centers, (0, 2, 1))
    d2 = jnp.sum(ctr ** 2, axis=-1)[:, :, None] + jnp.sum(pts ** 2, axis=-1)[:, None, :] - 2.0 * jnp.einsum('bmc,bnc->bmn', ctr, pts)
    mask = d2 < radius * radius
    order = jnp.argsort(jnp.where(mask, 0, 1).astype(jnp.int32), axis=-1)
    idx = order[:, :, :K]
    cnt = mask.sum(axis=-1)
    first = idx[:, :, 0:1]
    ar = jnp.arange(K)[None, None, :]
    return jnp.where(ar < jnp.maximum(cnt, 1)[:, :, None], idx, jnp.broadcast_to(first, idx.shape))


def _sa(p, features, coords, M, radius, K):
    cidx = _fps(jax.lax.stop_gradient(coords), M)
    centers = jax.vmap(lambda c, i: c[:, i])(coords, cidx)
    nidx = _ball_query(jax.lax.stop_gradient(coords), jax.lax.stop_gradient(centers), radius, K)
    gc = jax.vmap(lambda c, i: c[:, i])(coords, nidx) - centers[:, :, :, None]
    gf = jax.vmap(lambda f, i: f[:, i])(features, nidx)
    h = jnp.concatenate([gc, gf], axis=1)
    for lp in p:
        h = jnp.einsum('oc,bcmk->bomk', lp['w'], h) + lp['b'][None, :, None, None]
        h = jax.nn.relu(_bn(h, lp['g'], lp['be'], 1e-5))
    return jnp.max(h, axis=-1), centers


def _identity_pallas(x):
    return pl.pallas_call(
        lambda x_ref, o_ref: o_ref.__setitem__(slice(None), x_ref[...]),
        out_shape=jax.ShapeDtypeStruct(x.shape, x.dtype),
    )(x)


def kernel(inputs, params):
    coords = inputs[:, :3, :]
    features = inputs
    features = _pvconv(params['s1pv0'], features, coords, 32)
    features = _pvconv(params['s1pv1'], features, coords, 32)
    features, coords = _sa(params['s1sa'], features, coords, 1024, 0.1, 32)
    features = _pvconv(params['s2pv0'], features, coords, 16)
    features = _pvconv(params['s2pv1'], features, coords, 16)
    features, coords = _sa(params['s2sa'], features, coords, 256, 0.2, 32)
    features = _pvconv(params['s3pv0'], features, coords, 8)
    features = _pvconv(params['s3pv1'], features, coords, 8)
    features, coords = _sa(params['s3sa'], features, coords, 64, 0.4, 32)
    features, coords = _sa(params['s4sa'], features, coords, 16, 0.8, 32)
    features, coords = _sa(params['s5sa'], features, coords, 1, 0.16, 16)
    features = _identity_pallas(features)
    return features, coords


# speedup vs baseline: 1.4208x; 1.4208x over previous
import jax
import jax.numpy as jnp
from jax.experimental import pallas as pl
from jax.experimental.pallas import tpu as pltpu


def _fps_centers(coords, M):
    B, _, N = coords.shape

    def body(coords_ref, out_ref, dists_ref, last_ref):
        i = pl.program_id(0)

        @pl.when(i == 0)
        def _():
            dists_ref[...] = jnp.full((B, N), 1e10, jnp.float32)
            p0 = coords_ref[:, :, 0:1].reshape(B, 3)
            last_ref[...] = p0
            out_ref[...] = p0[None, :, :]

        @pl.when(i > 0)
        def _():
            x = coords_ref[:, 0, :]
            y = coords_ref[:, 1, :]
            z = coords_ref[:, 2, :]
            lx = last_ref[:, 0:1]
            ly = last_ref[:, 1:2]
            lz = last_ref[:, 2:3]
            d = (x - lx) ** 2 + (y - ly) ** 2 + (z - lz) ** 2
            dists = jnp.minimum(dists_ref[...], d)
            dists_ref[...] = dists
            m = jnp.max(dists, axis=1, keepdims=True)
            iota = jax.lax.broadcasted_iota(jnp.int32, (B, N), 1)
            sel = jnp.where(dists == m, iota, N)
            amin = jnp.min(sel, axis=1, keepdims=True)
            hit = iota == amin
            nx = jnp.sum(jnp.where(hit, x, 0.0), axis=1)
            ny = jnp.sum(jnp.where(hit, y, 0.0), axis=1)
            nz = jnp.sum(jnp.where(hit, z, 0.0), axis=1)
            nl = jnp.stack([nx, ny, nz], axis=1)
            last_ref[...] = nl
            out_ref[...] = nl[None, :, :]

    centers = pl.pallas_call(
        body,
        grid=(M,),
        in_specs=[pl.BlockSpec((B, 3, N), lambda i: (0, 0, 0))],
        out_specs=pl.BlockSpec((1, B, 3), lambda i: (i, 0, 0)),
        out_shape=jax.ShapeDtypeStruct((M, B, 3), jnp.float32),
        scratch_shapes=[pltpu.VMEM((B, N), jnp.float32),
                        pltpu.VMEM((B, 3), jnp.float32)],
    )(coords)
    return jnp.transpose(centers, (1, 2, 0))


def _bn(x, g, b, eps):
    axes = tuple(i for i in range(x.ndim) if i != 1)
    m = jnp.mean(x, axis=axes, keepdims=True)
    v = jnp.var(x, axis=axes, keepdims=True)
    sh = [1] * x.ndim
    sh[1] = -1
    return g.reshape(sh) * (x - m) / jnp.sqrt(v + eps) + b.reshape(sh)


def _conv3d(x, w, b):
    y = jax.lax.conv_general_dilated(x, w, (1, 1, 1), 'SAME',
                                     dimension_numbers=('NCDHW', 'OIDHW', 'NCDHW'))
    return y + b.reshape(1, -1, 1, 1, 1)


def _voxelize(features, coords, r):
    B, C, N = features.shape
    norm = coords - jnp.mean(coords, axis=2, keepdims=True)
    scale = jnp.max(jnp.sqrt(jnp.sum(norm ** 2, axis=1, keepdims=True)), axis=2, keepdims=True) * 2.0
    norm = norm / scale + 0.5
    norm = jnp.clip(norm * r, 0.0, r - 1.0)
    vi = jnp.round(norm).astype(jnp.int32)
    flat = (vi[:, 0] * r + vi[:, 1]) * r + vi[:, 2]
    def scat(f, fl):
        sums = jax.ops.segment_sum(f.T, fl, num_segments=r * r * r)
        cnt = jax.ops.segment_sum(jnp.ones((fl.shape[0],), jnp.float32), fl, num_segments=r * r * r)
        return (sums / jnp.maximum(cnt, 1.0)[:, None]).T
    vox = jax.vmap(scat)(features, flat).reshape(B, C, r, r, r)
    return vox, norm


def _devoxelize(vox, norm, r):
    B, C = vox.shape[0], vox.shape[1]
    N = norm.shape[2]
    f = vox.reshape(B, C, r * r * r)
    def corners(v):
        v0 = jnp.floor(v)
        i0 = jnp.clip(v0.astype(jnp.int32), 0, r - 1)
        i1 = jnp.clip(i0 + 1, 0, r - 1)
        w1 = v - v0
        return ((i0, 1.0 - w1), (i1, w1))
    xs = corners(norm[:, 0])
    ys = corners(norm[:, 1])
    zs = corners(norm[:, 2])
    out = jnp.zeros((B, C, N), jnp.float32)
    for xi, wx in xs:
        for yi, wy in ys:
            for zi, wz in zs:
                flat = (xi * r + yi) * r + zi
                g = jnp.take_along_axis(f, jnp.broadcast_to(flat[:, None, :], (B, C, N)), axis=2)
                out = out + g * (wx * wy * wz)[:, None, :]
    return out


def _pvconv(p, features, coords, r):
    vox, norm = _voxelize(features, coords, r)
    h = _conv3d(vox, p['vw1'], p['vb1'])
    h = jax.nn.leaky_relu(_bn(h, p['g1'], p['be1'], 1e-4), 0.1)
    h = _conv3d(h, p['vw2'], p['vb2'])
    h = jax.nn.leaky_relu(_bn(h, p['g2'], p['be2'], 1e-4), 0.1)
    s = jnp.mean(h, axis=(2, 3, 4))
    s = jax.nn.relu(s @ p['sew1'].T + p['seb1'])
    s = jax.nn.sigmoid(s @ p['sew2'].T + p['seb2'])
    h = h * s[:, :, None, None, None]
    vfeat = _devoxelize(h, norm, r)
    pf = jnp.einsum('oc,bcn->bon', p['pw'], features) + p['pb'][None, :, None]
    pf = jax.nn.relu(_bn(pf, p['pg'], p['pbe'], 1e-5))
    return vfeat + pf


def _fps(coords, M):
    B, _, N = coords.shape
    pts = jnp.transpose(coords, (0, 2, 1))
    def body(i, state):
        idxs, dists, last = state
        lp = jnp.take_along_axis(pts, last[:, None, None], axis=1)
        d = jnp.sum((pts - lp) ** 2, axis=-1)
        dists = jnp.minimum(dists, d)
        nxt = jnp.argmax(dists, axis=-1).astype(jnp.int32)
        idxs = idxs.at[:, i].set(nxt)
        return (idxs, dists, nxt)
    state = (jnp.zeros((B, M), jnp.int32), jnp.full((B, N), 1e10, jnp.float32), jnp.zeros((B,), jnp.int32))
    return jax.lax.fori_loop(1, M, body, state)[0]


def _ball_query(coords, centers, radius, K):
    pts = jnp.transpose(coords, (0, 2, 1))
    ctr = jnp.transpose(centers, (0, 2, 1))
    d2 = jnp.sum(ctr ** 2, axis=-1)[:, :, None] + jnp.sum(pts ** 2, axis=-1)[:, None, :] - 2.0 * jnp.einsum('bmc,bnc->bmn', ctr, pts)
    mask = d2 < radius * radius
    order = jnp.argsort(jnp.where(mask, 0, 1).astype(jnp.int32), axis=-1)
    idx = order[:, :, :K]
    cnt = mask.sum(axis=-1)
    first = idx[:, :, 0:1]
    ar = jnp.arange(K)[None, None, :]
    return jnp.where(ar < jnp.maximum(cnt, 1)[:, :, None], idx, jnp.broadcast_to(first, idx.shape))


def _sa(p, features, coords, M, radius, K):
    centers = _fps_centers(coords, M)
    nidx = _ball_query(jax.lax.stop_gradient(coords), jax.lax.stop_gradient(centers), radius, K)
    gc = jax.vmap(lambda c, i: c[:, i])(coords, nidx) - centers[:, :, :, None]
    gf = jax.vmap(lambda f, i: f[:, i])(features, nidx)
    h = jnp.concatenate([gc, gf], axis=1)
    for lp in p:
        h = jnp.einsum('oc,bcmk->bomk', lp['w'], h) + lp['b'][None, :, None, None]
        h = jax.nn.relu(_bn(h, lp['g'], lp['be'], 1e-5))
    return jnp.max(h, axis=-1), centers


def _identity_pallas(x):
    return pl.pallas_call(
        lambda x_ref, o_ref: o_ref.__setitem__(slice(None), x_ref[...]),
        out_shape=jax.ShapeDtypeStruct(x.shape, x.dtype),
    )(x)


def kernel(inputs, params):
    coords = inputs[:, :3, :]
    features = inputs
    features = _pvconv(params['s1pv0'], features, coords, 32)
    features = _pvconv(params['s1pv1'], features, coords, 32)
    features, coords = _sa(params['s1sa'], features, coords, 1024, 0.1, 32)
    features = _pvconv(params['s2pv0'], features, coords, 16)
    features = _pvconv(params['s2pv1'], features, coords, 16)
    features, coords = _sa(params['s2sa'], features, coords, 256, 0.2, 32)
    features = _pvconv(params['s3pv0'], features, coords, 8)
    features = _pvconv(params['s3pv1'], features, coords, 8)
    features, coords = _sa(params['s3sa'], features, coords, 64, 0.4, 32)
    features, coords = _sa(params['s4sa'], features, coords, 16, 0.8, 32)
    features, coords = _sa(params['s5sa'], features, coords, 1, 0.16, 16)
    features = _identity_pallas(features)
    return features, coords
